# baseline (speedup 1.0000x reference)
import functools

import jax
import jax.numpy as jnp
from jax import lax
from jax.experimental import pallas as pl
from jax.experimental.pallas import tpu as pltpu
from jax.experimental.pallas import tpu_sc as plsc

N_CLASSES = 1000000
M_DIM = 32
BATCH = 16384

NUM_CORES = 2
NUM_SUBCORES = 16
LANES = 16
NUM_WORKERS = NUM_CORES * NUM_SUBCORES
B_PER_W = BATCH // NUM_WORKERS
CHUNK = 8
N_PAIRS = B_PER_W // LANES


def _rsqrt_newton(s):
  i = lax.bitcast_convert_type(s, jnp.int32)
  i = jnp.int32(0x5F3759DF) - lax.shift_right_logical(i, 1)
  y = lax.bitcast_convert_type(i, jnp.float32)
  half = s * 0.5
  for _ in range(3):
    y = y * (1.5 - half * y * y)
  return y


@jax.jit
def _embed_norm_t(idx, table_t):
  mesh = plsc.VectorSubcoreMesh(core_axis_name="c", subcore_axis_name="s")

  @functools.partial(
      pl.kernel,
      out_type=jax.ShapeDtypeStruct((M_DIM, BATCH), jnp.float32),
      mesh=mesh,
      scratch_types=[
          pltpu.VMEM((B_PER_W,), jnp.int32),
          pltpu.VMEM((CHUNK, M_DIM, 128), jnp.float32),
          pltpu.VMEM((CHUNK, M_DIM, 128), jnp.float32),
          pltpu.VMEM((M_DIM, B_PER_W), jnp.float32),
          pltpu.SemaphoreType.DMA,
          pltpu.SemaphoreType.DMA,
      ],
      compiler_params=pltpu.CompilerParams(
          needs_layout_passes=False, use_tc_tiling_on_sc=True
      ),
  )
  def k(idx_hbm, table_hbm, out_hbm, idx_v, blk_a, blk_b, out_v, sem_a, sem_b):
    wid = lax.axis_index("s") * NUM_CORES + lax.axis_index("c")
    base = wid * B_PER_W
    pltpu.sync_copy(idx_hbm.at[pl.ds(base, B_PER_W)], idx_v)
    lane_ids = lax.iota(jnp.int32, LANES)

    def issue(buf, sem, p, h):
      iv = idx_v[pl.ds(p * LANES, LANES)]
      for l in range(CHUNK):
        off = pl.multiple_of(
            lax.shift_right_logical(iv[h * CHUNK + l], 7) * 128, 128
        )
        for q in range(4):
          pltpu.async_copy(
              table_hbm.at[pl.ds(q * 8, 8), pl.ds(off, 128)],
              buf.at[l, pl.ds(q * 8, 8)],
              sem,
          )

    def drain(buf, sem):
      for l in range(CHUNK):
        for q in range(4):
          pltpu.make_async_copy(
              table_hbm.at[pl.ds(0, 8), pl.ds(0, 128)],
              buf.at[l, pl.ds(q * 8, 8)],
              sem,
          ).wait()

    def compute(buf, p, h):
      iv = idx_v[pl.ds(p * LANES, LANES)]
      sub = iv & 127
      for l in range(CHUNK):
        j = jnp.broadcast_to(sub[h * CHUNK + l], (LANES,))
        rsel = jnp.broadcast_to(jnp.int32(l), (LANES,))
        v0 = plsc.load_gather(buf, [rsel, lane_ids, j])
        v1 = plsc.load_gather(buf, [rsel, lane_ids + LANES, j])
        ss = v0 * v0 + v1 * v1
        tot = jnp.broadcast_to(jnp.sum(ss), (LANES,))
        y = _rsqrt_newton(tot)
        col = jnp.broadcast_to(p * LANES + h * CHUNK + l, (LANES,))
        plsc.store_scatter(out_v, [lane_ids, col], v0 * y)
        plsc.store_scatter(out_v, [lane_ids + LANES, col], v1 * y)

    issue(blk_a, sem_a, 0, 0)

    @pl.loop(0, N_PAIRS)
    def _(i):
      issue(blk_b, sem_b, i, 1)
      drain(blk_a, sem_a)
      compute(blk_a, i, 0)

      @pl.when(i < N_PAIRS - 1)
      def _():
        issue(blk_a, sem_a, i + 1, 0)

      drain(blk_b, sem_b)
      compute(blk_b, i, 1)

    pltpu.sync_copy(out_v, out_hbm.at[:, pl.ds(base, B_PER_W)])

  return k(idx, table_t)


def kernel(x, table):
  out_t = _embed_norm_t(x.astype(jnp.int32), table.T)
  return out_t.T


# speedup vs baseline: 1.0023x; 1.0023x over previous
import functools

import jax
import jax.numpy as jnp
from jax import lax
from jax.experimental import pallas as pl
from jax.experimental.pallas import tpu as pltpu
from jax.experimental.pallas import tpu_sc as plsc

N_CLASSES = 1000000
M_DIM = 32
BATCH = 16384

NUM_CORES = 2
NUM_SUBCORES = 16
LANES = 16
NUM_WORKERS = NUM_CORES * NUM_SUBCORES
B_PER_W = BATCH // NUM_WORKERS
CHUNK = 8
N_PAIRS = B_PER_W // LANES


def _rsqrt_newton(s):
  i = lax.bitcast_convert_type(s, jnp.int32)
  i = jnp.int32(0x5F3759DF) - lax.shift_right_logical(i, 1)
  y = lax.bitcast_convert_type(i, jnp.float32)
  half = s * 0.5
  for _ in range(3):
    y = y * (1.5 - half * y * y)
  return y


@jax.jit
def _embed_norm_t(idx, table_t):
  mesh = plsc.VectorSubcoreMesh(core_axis_name="c", subcore_axis_name="s")

  @functools.partial(
      pl.kernel,
      out_type=jax.ShapeDtypeStruct((M_DIM, BATCH), jnp.float32),
      mesh=mesh,
      scratch_types=[
          pltpu.VMEM((B_PER_W,), jnp.int32),
          pltpu.VMEM((CHUNK, M_DIM, 128), jnp.float32),
          pltpu.VMEM((CHUNK, M_DIM, 128), jnp.float32),
          pltpu.VMEM((M_DIM, B_PER_W), jnp.float32),
          pltpu.SemaphoreType.DMA,
          pltpu.SemaphoreType.DMA,
      ],
      compiler_params=pltpu.CompilerParams(
          needs_layout_passes=False, use_tc_tiling_on_sc=True
      ),
  )
  def k(idx_hbm, table_hbm, out_hbm, idx_v, blk_a, blk_b, out_v, sem_a, sem_b):
    wid = lax.axis_index("s") * NUM_CORES + lax.axis_index("c")
    base = wid * B_PER_W
    pltpu.sync_copy(idx_hbm.at[pl.ds(base, B_PER_W)], idx_v)
    lane_ids = lax.iota(jnp.int32, LANES)

    def issue(buf, sem, p, h):
      iv = idx_v[pl.ds(p * LANES, LANES)]
      for l in range(CHUNK):
        off = pl.multiple_of(
            lax.shift_right_logical(iv[h * CHUNK + l], 7) * 128, 128
        )
        pltpu.async_copy(table_hbm.at[:, pl.ds(off, 128)], buf.at[l], sem)

    def drain(buf, sem):
      for l in range(CHUNK):
        pltpu.make_async_copy(
            table_hbm.at[:, pl.ds(0, 128)], buf.at[l], sem
        ).wait()

    def compute(buf, p, h):
      iv = idx_v[pl.ds(p * LANES, LANES)]
      sub = iv & 127
      for l in range(CHUNK):
        j = jnp.broadcast_to(sub[h * CHUNK + l], (LANES,))
        rsel = jnp.broadcast_to(jnp.int32(l), (LANES,))
        v0 = plsc.load_gather(buf, [rsel, lane_ids, j])
        v1 = plsc.load_gather(buf, [rsel, lane_ids + LANES, j])
        ss = v0 * v0 + v1 * v1
        tot = jnp.broadcast_to(jnp.sum(ss), (LANES,))
        y = _rsqrt_newton(tot)
        col = jnp.broadcast_to(p * LANES + h * CHUNK + l, (LANES,))
        plsc.store_scatter(out_v, [lane_ids, col], v0 * y)
        plsc.store_scatter(out_v, [lane_ids + LANES, col], v1 * y)

    issue(blk_a, sem_a, 0, 0)

    @pl.loop(0, N_PAIRS)
    def _(i):
      issue(blk_b, sem_b, i, 1)
      drain(blk_a, sem_a)
      compute(blk_a, i, 0)

      @pl.when(i < N_PAIRS - 1)
      def _():
        issue(blk_a, sem_a, i + 1, 0)

      drain(blk_b, sem_b)
      compute(blk_b, i, 1)

    pltpu.sync_copy(out_v, out_hbm.at[:, pl.ds(base, B_PER_W)])

  return k(idx, table_t)


def kernel(x, table):
  out_t = _embed_norm_t(x.astype(jnp.int32), table.T)
  return out_t.T


# speedup vs baseline: 1.1353x; 1.1327x over previous
import functools

import jax
import jax.numpy as jnp
from jax import lax
from jax.experimental import pallas as pl
from jax.experimental.pallas import tpu as pltpu
from jax.experimental.pallas import tpu_sc as plsc

N_CLASSES = 1000000
M_DIM = 32
BATCH = 16384

NUM_CORES = 2
NUM_SUBCORES = 16
LANES = 16
NUM_WORKERS = NUM_CORES * NUM_SUBCORES
B_SC = 12288
B_TC = BATCH - B_SC
B_PER_W = B_SC // NUM_WORKERS
CHUNK = 8
N_PAIRS = B_PER_W // LANES
TCH = 512
TC_STEPS = B_TC // TCH


def _rsqrt_newton(s):
  i = lax.bitcast_convert_type(s, jnp.int32)
  i = jnp.int32(0x5F3759DF) - lax.shift_right_logical(i, 1)
  y = lax.bitcast_convert_type(i, jnp.float32)
  half = s * 0.5
  for _ in range(3):
    y = y * (1.5 - half * y * y)
  return y


@jax.jit
def _embed_norm_t(idx, table_t):
  mesh = plsc.VectorSubcoreMesh(core_axis_name="c", subcore_axis_name="s")

  @functools.partial(
      pl.kernel,
      out_type=jax.ShapeDtypeStruct((M_DIM, B_SC), jnp.float32),
      mesh=mesh,
      scratch_types=[
          pltpu.VMEM((B_PER_W,), jnp.int32),
          pltpu.VMEM((CHUNK, M_DIM, 128), jnp.float32),
          pltpu.VMEM((CHUNK, M_DIM, 128), jnp.float32),
          pltpu.VMEM((M_DIM, B_PER_W), jnp.float32),
          pltpu.SemaphoreType.DMA,
          pltpu.SemaphoreType.DMA,
      ],
      compiler_params=pltpu.CompilerParams(
          needs_layout_passes=False, use_tc_tiling_on_sc=True
      ),
  )
  def k(idx_hbm, table_hbm, out_hbm, idx_v, blk_a, blk_b, out_v, sem_a, sem_b):
    wid = lax.axis_index("s") * NUM_CORES + lax.axis_index("c")
    base = wid * B_PER_W
    pltpu.sync_copy(idx_hbm.at[pl.ds(base, B_PER_W)], idx_v)
    lane_ids = lax.iota(jnp.int32, LANES)

    def issue(buf, sem, p, h):
      iv = idx_v[pl.ds(p * LANES, LANES)]
      for l in range(CHUNK):
        off = pl.multiple_of(
            lax.shift_right_logical(iv[h * CHUNK + l], 7) * 128, 128
        )
        pltpu.async_copy(table_hbm.at[:, pl.ds(off, 128)], buf.at[l], sem)

    def drain(buf, sem):
      for l in range(CHUNK):
        pltpu.make_async_copy(
            table_hbm.at[:, pl.ds(0, 128)], buf.at[l], sem
        ).wait()

    def compute(buf, p, h):
      iv = idx_v[pl.ds(p * LANES, LANES)]
      sub = iv & 127
      for l in range(CHUNK):
        j = jnp.broadcast_to(sub[h * CHUNK + l], (LANES,))
        rsel = jnp.broadcast_to(jnp.int32(l), (LANES,))
        v0 = plsc.load_gather(buf, [rsel, lane_ids, j])
        v1 = plsc.load_gather(buf, [rsel, lane_ids + LANES, j])
        ss = v0 * v0 + v1 * v1
        tot = jnp.broadcast_to(jnp.sum(ss), (LANES,))
        y = _rsqrt_newton(tot)
        col = jnp.broadcast_to(p * LANES + h * CHUNK + l, (LANES,))
        plsc.store_scatter(out_v, [lane_ids, col], v0 * y)
        plsc.store_scatter(out_v, [lane_ids + LANES, col], v1 * y)

    issue(blk_a, sem_a, 0, 0)

    @pl.loop(0, N_PAIRS)
    def _(i):
      issue(blk_b, sem_b, i, 1)
      drain(blk_a, sem_a)
      compute(blk_a, i, 0)

      @pl.when(i < N_PAIRS - 1)
      def _():
        issue(blk_a, sem_a, i + 1, 0)

      drain(blk_b, sem_b)
      compute(blk_b, i, 1)

    pltpu.sync_copy(out_v, out_hbm.at[:, pl.ds(base, B_PER_W)])

  return k(idx, table_t)


def _tc_body(idx_s, tab_ref, oh_ref, out_ref, blk, sems):
  g = pl.program_id(0)

  def issue(buf_i, step):
    def one(i, carry):
      ival = idx_s[step * TCH + i]
      off = pl.multiple_of(lax.shift_right_logical(ival, 7) * 128, 128)
      pltpu.make_async_copy(
          tab_ref.at[:, pl.ds(off, 128)], blk.at[buf_i, i], sems.at[buf_i]
      ).start()
      return carry

    lax.fori_loop(0, TCH, one, 0)

  @pl.when(g == 0)
  def _():
    issue(0, 0)

  @pl.when(g + 1 < TC_STEPS)
  def _():
    issue((g + 1) % 2, g + 1)

  b = g % 2

  def wait_one(i, carry):
    pltpu.make_async_copy(
        tab_ref.at[:, pl.ds(0, 128)], blk.at[b, i], sems.at[b]
    ).wait()
    return carry

  lax.fori_loop(0, TCH, wait_one, 0)

  data = blk[b]
  oh = oh_ref[...]
  sel = jnp.sum(data * oh[:, None, :], axis=2)
  ss = jnp.sum(sel * sel, axis=1, keepdims=True)
  out_ref[...] = sel * lax.rsqrt(ss)


def _embed_norm_tc(idx, table_t, onehot):
  grid_spec = pltpu.PrefetchScalarGridSpec(
      num_scalar_prefetch=1,
      grid=(TC_STEPS,),
      in_specs=[
          pl.BlockSpec(memory_space=pltpu.MemorySpace.HBM),
          pl.BlockSpec((TCH, 128), lambda g, idx_ref: (g, 0)),
      ],
      out_specs=pl.BlockSpec((TCH, M_DIM), lambda g, idx_ref: (g, 0)),
      scratch_shapes=[
          pltpu.VMEM((2, TCH, M_DIM, 128), jnp.float32),
          pltpu.SemaphoreType.DMA((2,)),
      ],
  )
  return pl.pallas_call(
      _tc_body,
      grid_spec=grid_spec,
      out_shape=jax.ShapeDtypeStruct((B_TC, M_DIM), jnp.float32),
      compiler_params=pltpu.CompilerParams(
          vmem_limit_bytes=50 * 1024 * 1024
      ),
  )(idx, table_t, onehot)


@jax.jit
def _embed_norm_full(idx, table_t):
  idx_tc = idx[B_SC:]
  onehot = (
      (idx_tc[:, None] & 127) == lax.broadcasted_iota(jnp.int32, (1, 128), 1)
  ).astype(jnp.float32)
  sc = _embed_norm_t(idx[:B_SC], table_t)
  tc = _embed_norm_tc(idx_tc, table_t, onehot)
  return jnp.concatenate([sc, tc.T], axis=1)


def kernel(x, table):
  out_t = _embed_norm_full(x.astype(jnp.int32), table.T)
  return out_t.T
